# baseline (speedup 1.0000x reference)
import functools

import jax
import jax.numpy as jnp
from jax import lax
from jax.experimental import pallas as pl
from jax.experimental.pallas import tpu as pltpu
from jax.experimental.pallas import tpu_sc as plsc

N = 10000
E = 320000
D = 128
T = 3

NC = 2
NS = 16
NW = NC * NS
CHUNK = 120
NCHUNK = 88
NPHASE = 11
PCHUNK = NCHUNK // NPHASE
EPW = CHUNK * NCHUNK
EPAD = EPW * NW
RPT = 624
TAIL = N - NS * RPT

BN = 1000
NB = N // BN
NR = N + BN


def _transform_body(attn_ref, x_ref, w_ref, b_ref, o_ref):
    t = pl.program_id(0)
    a0, a1, a2, a3 = attn_ref[0], attn_ref[1], attn_ref[2], attn_ref[3]
    m = jnp.maximum(jnp.maximum(a0, a1), jnp.maximum(a2, a3))
    denom = (jnp.exp(a0 - m) + jnp.exp(a1 - m)
             + jnp.exp(a2 - m) + jnp.exp(a3 - m))
    wt = jnp.exp(attn_ref[t] - m) / denom
    y = lax.dot_general(x_ref[...], w_ref[0], (((1,), (1,)), ((), ())),
                        preferred_element_type=jnp.float32)
    val = (y + b_ref[0]) * wt
    o_ref[0] = jnp.where(pl.program_id(1) < NB, val, 0.0)


def _transform(attn, x, wstack, bstack):
    return pl.pallas_call(
        _transform_body,
        grid=(T + 1, NB + 1),
        in_specs=[
            pl.BlockSpec(memory_space=pltpu.SMEM),
            pl.BlockSpec((BN, D), lambda t, j: (jnp.minimum(j, NB - 1), 0)),
            pl.BlockSpec((1, D, D), lambda t, j: (t, 0, 0)),
            pl.BlockSpec((1, 1, D), lambda t, j: (t, 0, 0)),
        ],
        out_specs=pl.BlockSpec((1, BN, D), lambda t, j: (t, j, 0)),
        out_shape=jax.ShapeDtypeStruct((T + 1, NR, D), jnp.float32),
    )(attn, x, wstack, bstack)


def _edge_body(g_hbm, gidx_hbm, didx_hbm, zeros_hbm, out_hbm,
               gidx_v, didx_v, rows0_v, rows1_v, rows2_v,
               sem0, sem1, sem2, acc_sh):
    cid = lax.axis_index("c")
    sid = lax.axis_index("s")
    wid = cid * NS + sid

    pltpu.sync_copy(zeros_hbm.at[pl.ds(sid * RPT, RPT)],
                    acc_sh.at[pl.ds(sid * RPT, RPT)])

    @pl.when(sid == NS - 1)
    def _():
        pltpu.sync_copy(zeros_hbm.at[pl.ds(NS * RPT, TAIL)],
                        acc_sh.at[pl.ds(NS * RPT, TAIL)])

    plsc.subcore_barrier()

    for phase in range(NPHASE):
        pltpu.sync_copy(gidx_hbm.at[wid, pl.ds(phase * PCHUNK, PCHUNK)],
                        gidx_v)
        pltpu.sync_copy(didx_hbm.at[wid, pl.ds(phase * PCHUNK, PCHUNK)],
                        didx_v)

        bufs = ((rows0_v, sem0), (rows1_v, sem1), (rows2_v, sem2))
        for b in range(3):
            rv, sm = bufs[b]
            pltpu.async_copy(g_hbm.at[gidx_v.at[b]], rv, sm)
        for j in range(PCHUNK):
            rv, sm = bufs[j % 3]
            pltpu.make_async_copy(g_hbm.at[gidx_v.at[j]], rv, sm).wait()
            pltpu.sync_copy(rv, acc_sh.at[didx_v.at[j]], add=True)
            if j + 3 < PCHUNK:
                pltpu.async_copy(g_hbm.at[gidx_v.at[j + 3]], rv, sm)

    plsc.subcore_barrier()
    pltpu.sync_copy(acc_sh.at[pl.ds(sid * RPT, RPT)],
                    out_hbm.at[cid, pl.ds(sid * RPT, RPT)])

    @pl.when(sid == NS - 1)
    def _():
        pltpu.sync_copy(acc_sh.at[pl.ds(NS * RPT, TAIL)],
                        out_hbm.at[cid, pl.ds(NS * RPT, TAIL)])


def _edge_pass(g, gidx3, didx3, zeros):
    mesh = plsc.VectorSubcoreMesh(core_axis_name="c", subcore_axis_name="s")
    run = pl.kernel(
        _edge_body,
        out_type=jax.ShapeDtypeStruct((NC, N, D), jnp.float32),
        mesh=mesh,
        scratch_types=[
            pltpu.VMEM((PCHUNK, CHUNK), jnp.int32),
            pltpu.VMEM((PCHUNK, CHUNK), jnp.int32),
            pltpu.VMEM((CHUNK, D), jnp.float32),
            pltpu.VMEM((CHUNK, D), jnp.float32),
            pltpu.VMEM((CHUNK, D), jnp.float32),
            pltpu.SemaphoreType.DMA,
            pltpu.SemaphoreType.DMA,
            pltpu.SemaphoreType.DMA,
            pltpu.VMEM_SHARED((N, D), jnp.float32),
        ],
    )
    return run(g, gidx3, didx3, zeros)


def _combine_body(acc_ref, g_ref, o_ref):
    y = acc_ref[0] + acc_ref[1] + g_ref[0]
    o_ref[...] = 0.5 * y * (1.0 + lax.erf(y * 0.7071067811865476))


def _combine(acc, g):
    return pl.pallas_call(
        _combine_body,
        grid=(N // BN,),
        in_specs=[
            pl.BlockSpec((NC, BN, D), lambda j: (0, j, 0)),
            pl.BlockSpec((1, BN, D), lambda j: (T, j, 0)),
        ],
        out_specs=pl.BlockSpec((BN, D), lambda j: (j, 0)),
        out_shape=jax.ShapeDtypeStruct((N, D), jnp.float32),
    )(acc, g)


@jax.jit
def kernel(x, edge_index, edge_types, W0, b0, W1, b1, W2, b2, Ws, bs, attn):
    wstack = jnp.stack([W0, W1, W2, Ws])
    bstack = jnp.stack([b0, b1, b2, bs]).reshape(T + 1, 1, D)

    g = _transform(attn, x, wstack, bstack)

    src = edge_index[0]
    dst = edge_index[1]
    pad = EPAD - E
    ii = jnp.arange(pad, dtype=jnp.int32)
    gidx = jnp.concatenate(
        [(edge_types * NR + src).astype(jnp.int32),
         (ii % 4) * NR + N + (ii // 4) % BN])
    didx = jnp.concatenate(
        [dst.astype(jnp.int32), jnp.arange(pad, dtype=jnp.int32) % N])
    gidx3 = gidx.reshape(NW, NCHUNK, CHUNK)
    didx3 = didx.reshape(NW, NCHUNK, CHUNK)
    zeros = jnp.zeros((N, D), jnp.float32)

    acc = _edge_pass(g.reshape((T + 1) * NR, D), gidx3, didx3, zeros)
    return _combine(acc, g)


# speedup vs baseline: 1.1303x; 1.1303x over previous
import functools

import jax
import jax.numpy as jnp
from jax import lax
from jax.experimental import pallas as pl
from jax.experimental.pallas import tpu as pltpu
from jax.experimental.pallas import tpu_sc as plsc

N = 10000
E = 320000
D = 128
T = 3

NC = 2
NS = 16
NW = NC * NS
CHUNK = 128
NCHUNK = 80
NPHASE = 2
PCHUNK = NCHUNK // NPHASE
EPW = CHUNK * NCHUNK
EPAD = EPW * NW
RPT = 624
TAIL = N - NS * RPT

BN = 1000
NB = N // BN
NR = N + BN


def _transform_body(attn_ref, x_ref, w_ref, b_ref, o_ref):
    t = pl.program_id(0)
    a0, a1, a2, a3 = attn_ref[0], attn_ref[1], attn_ref[2], attn_ref[3]
    m = jnp.maximum(jnp.maximum(a0, a1), jnp.maximum(a2, a3))
    denom = (jnp.exp(a0 - m) + jnp.exp(a1 - m)
             + jnp.exp(a2 - m) + jnp.exp(a3 - m))
    wt = jnp.exp(attn_ref[t] - m) / denom
    y = lax.dot_general(x_ref[...], w_ref[0], (((1,), (1,)), ((), ())),
                        preferred_element_type=jnp.float32)
    val = (y + b_ref[0]) * wt
    o_ref[0] = jnp.where(pl.program_id(1) < NB, val, 0.0)


def _transform(attn, x, wstack, bstack):
    return pl.pallas_call(
        _transform_body,
        grid=(T + 1, NB + 1),
        in_specs=[
            pl.BlockSpec(memory_space=pltpu.SMEM),
            pl.BlockSpec((BN, D), lambda t, j: (jnp.minimum(j, NB - 1), 0)),
            pl.BlockSpec((1, D, D), lambda t, j: (t, 0, 0)),
            pl.BlockSpec((1, 1, D), lambda t, j: (t, 0, 0)),
        ],
        out_specs=pl.BlockSpec((1, BN, D), lambda t, j: (t, j, 0)),
        out_shape=jax.ShapeDtypeStruct((T + 1, NR, D), jnp.float32),
    )(attn, x, wstack, bstack)


def _edge_body(g_hbm, gidx_hbm, didx_hbm, zeros_hbm, out_hbm,
               gidx_v, didx_v, rows0_v, rows1_v, sem0, sem1, acc_sh):
    cid = lax.axis_index("c")
    sid = lax.axis_index("s")
    wid = cid * NS + sid

    pltpu.sync_copy(zeros_hbm.at[pl.ds(sid * RPT, RPT)],
                    acc_sh.at[pl.ds(sid * RPT, RPT)])

    @pl.when(sid == NS - 1)
    def _():
        pltpu.sync_copy(zeros_hbm.at[pl.ds(NS * RPT, TAIL)],
                        acc_sh.at[pl.ds(NS * RPT, TAIL)])

    plsc.subcore_barrier()

    for phase in range(NPHASE):
        pltpu.sync_copy(gidx_hbm.at[wid, pl.ds(phase * PCHUNK, PCHUNK)],
                        gidx_v)
        pltpu.sync_copy(didx_hbm.at[wid, pl.ds(phase * PCHUNK, PCHUNK)],
                        didx_v)

        pltpu.async_copy(g_hbm.at[gidx_v.at[0]], rows0_v, sem0)
        pltpu.async_copy(g_hbm.at[gidx_v.at[1]], rows1_v, sem1)

        def pair(i, carry):
            j = 2 * i
            pltpu.make_async_copy(g_hbm.at[gidx_v.at[j]], rows0_v,
                                  sem0).wait()
            pltpu.sync_copy(rows0_v, acc_sh.at[didx_v.at[j]], add=True)

            @pl.when(j + 2 < PCHUNK)
            def _():
                pltpu.async_copy(g_hbm.at[gidx_v.at[j + 2]], rows0_v, sem0)

            pltpu.make_async_copy(g_hbm.at[gidx_v.at[j + 1]], rows1_v,
                                  sem1).wait()
            pltpu.sync_copy(rows1_v, acc_sh.at[didx_v.at[j + 1]], add=True)

            @pl.when(j + 3 < PCHUNK)
            def _():
                pltpu.async_copy(g_hbm.at[gidx_v.at[j + 3]], rows1_v, sem1)

            return carry

        lax.fori_loop(0, PCHUNK // 2, pair, 0, unroll=False)

    plsc.subcore_barrier()
    pltpu.sync_copy(acc_sh.at[pl.ds(sid * RPT, RPT)],
                    out_hbm.at[cid, pl.ds(sid * RPT, RPT)])

    @pl.when(sid == NS - 1)
    def _():
        pltpu.sync_copy(acc_sh.at[pl.ds(NS * RPT, TAIL)],
                        out_hbm.at[cid, pl.ds(NS * RPT, TAIL)])


def _edge_pass(g, gidx3, didx3, zeros):
    mesh = plsc.VectorSubcoreMesh(core_axis_name="c", subcore_axis_name="s")
    run = pl.kernel(
        _edge_body,
        out_type=jax.ShapeDtypeStruct((NC, N, D), jnp.float32),
        mesh=mesh,
        scratch_types=[
            pltpu.VMEM((PCHUNK, CHUNK), jnp.int32),
            pltpu.VMEM((PCHUNK, CHUNK), jnp.int32),
            pltpu.VMEM((CHUNK, D), jnp.float32),
            pltpu.VMEM((CHUNK, D), jnp.float32),
            pltpu.SemaphoreType.DMA,
            pltpu.SemaphoreType.DMA,
            pltpu.VMEM_SHARED((N, D), jnp.float32),
        ],
    )
    return run(g, gidx3, didx3, zeros)


def _combine_body(acc_ref, g_ref, o_ref):
    y = acc_ref[0] + acc_ref[1] + g_ref[0]
    o_ref[...] = 0.5 * y * (1.0 + lax.erf(y * 0.7071067811865476))


def _combine(acc, g):
    return pl.pallas_call(
        _combine_body,
        grid=(N // BN,),
        in_specs=[
            pl.BlockSpec((NC, BN, D), lambda j: (0, j, 0)),
            pl.BlockSpec((1, BN, D), lambda j: (T, j, 0)),
        ],
        out_specs=pl.BlockSpec((BN, D), lambda j: (j, 0)),
        out_shape=jax.ShapeDtypeStruct((N, D), jnp.float32),
    )(acc, g)


@jax.jit
def kernel(x, edge_index, edge_types, W0, b0, W1, b1, W2, b2, Ws, bs, attn):
    wstack = jnp.stack([W0, W1, W2, Ws])
    bstack = jnp.stack([b0, b1, b2, bs]).reshape(T + 1, 1, D)

    g = _transform(attn, x, wstack, bstack)

    src = edge_index[0]
    dst = edge_index[1]
    pad = EPAD - E
    ii = jnp.arange(pad, dtype=jnp.int32)
    gidx = jnp.concatenate(
        [(edge_types * NR + src).astype(jnp.int32),
         (ii % 4) * NR + N + (ii // 4) % BN])
    didx = jnp.concatenate(
        [dst.astype(jnp.int32), jnp.arange(pad, dtype=jnp.int32) % N])
    gidx3 = gidx.reshape(NW, NCHUNK, CHUNK)
    didx3 = didx.reshape(NW, NCHUNK, CHUNK)
    zeros = jnp.zeros((N, D), jnp.float32)

    acc = _edge_pass(g.reshape((T + 1) * NR, D), gidx3, didx3, zeros)
    return _combine(acc, g)
